# baseline (speedup 1.0000x reference)
import functools

import jax
import jax.numpy as jnp
from jax import lax
from jax.experimental import pallas as pl
from jax.experimental.pallas import tpu as pltpu
from jax.experimental.pallas import tpu_sc as plsc

_NC = 2
_NS = 16
_NW = _NC * _NS


def _emb_call(n_chunks, chunk, D, idx, table):
    mesh = plsc.VectorSubcoreMesh(core_axis_name="c", subcore_axis_name="s")
    N = _NW * n_chunks * chunk

    @functools.partial(
        pl.kernel,
        mesh=mesh,
        compiler_params=pltpu.CompilerParams(use_tc_tiling_on_sc=False),
        out_type=jax.ShapeDtypeStruct((N, D), jnp.float32),
        scratch_types=[
            pltpu.VMEM((n_chunks, chunk), jnp.int32),
            pltpu.VMEM((2, chunk, D), jnp.float32),
            pltpu.SemaphoreType.DMA,
            pltpu.SemaphoreType.DMA,
            pltpu.SemaphoreType.DMA,
        ],
    )
    def emb(idx_hbm, table_hbm, out_hbm, idx_v, rows_v, isem, gsem, osem):
        wid = lax.axis_index("s") * _NC + lax.axis_index("c")
        base = wid * (n_chunks * chunk)
        pltpu.async_copy(idx_hbm.at[wid], idx_v, isem).wait()

        def gather(j, slot):
            return pltpu.async_copy(
                table_hbm.at[idx_v.at[j]], rows_v.at[slot], gsem)

        def put(j, slot):
            return pltpu.async_copy(
                rows_v.at[slot], out_hbm.at[pl.ds(base + j * chunk, chunk)],
                osem)

        gather(0, 0).wait()

        def body(j, _):
            slot = lax.rem(j, 2)
            nslot = 1 - slot
            g = gather(j + 1, nslot)
            put(j, slot).wait()
            g.wait()
            return 0

        lax.fori_loop(0, n_chunks - 1, body, 0, unroll=False)
        last = n_chunks - 1
        put(last, lax.rem(last, 2)).wait()

    return emb(idx, table)


def kernel(tokens, W_E):
    B, S = tokens.shape
    V, D = W_E.shape
    N = B * S
    chunk = 128
    n_chunks = N // (_NW * chunk)
    assert N == _NW * n_chunks * chunk
    idx = tokens.reshape(_NW, n_chunks, chunk).astype(jnp.int32)
    out = _emb_call(n_chunks, chunk, D, idx, W_E)
    return out.reshape(B, S, D)


# speedup vs baseline: 1.1183x; 1.1183x over previous
import functools

import jax
import jax.numpy as jnp
from jax import lax
from jax.experimental import pallas as pl
from jax.experimental.pallas import tpu as pltpu
from jax.experimental.pallas import tpu_sc as plsc

_NC = 2
_NS = 16
_NW = _NC * _NS


def _emb_call(n_chunks, chunk, D, idx, table):
    mesh = plsc.VectorSubcoreMesh(core_axis_name="c", subcore_axis_name="s")
    N = _NW * n_chunks * chunk

    @functools.partial(
        pl.kernel,
        mesh=mesh,
        compiler_params=pltpu.CompilerParams(use_tc_tiling_on_sc=False),
        out_type=jax.ShapeDtypeStruct((N, D), jnp.float32),
        scratch_types=[
            pltpu.VMEM((n_chunks, chunk), jnp.int32),
            pltpu.VMEM((2, chunk, D), jnp.float32),
            pltpu.SemaphoreType.DMA,
            pltpu.SemaphoreType.DMA,
            pltpu.SemaphoreType.DMA,
        ],
    )
    def emb(idx_hbm, table_hbm, out_hbm, idx_v, rows_v, isem, gsem, osem):
        wid = lax.axis_index("s") * _NC + lax.axis_index("c")
        base = wid * (n_chunks * chunk)
        pltpu.async_copy(idx_hbm.at[wid], idx_v, isem).wait()

        def gather(j, slot):
            return pltpu.async_copy(
                table_hbm.at[idx_v.at[j]], rows_v.at[slot], gsem)

        def put(j, slot):
            return pltpu.async_copy(
                rows_v.at[slot], out_hbm.at[pl.ds(base + j * chunk, chunk)],
                osem)

        gather(0, 0).wait()

        def body(j, _):
            slot = lax.rem(j, 2)
            nslot = 1 - slot
            g = gather(j + 1, nslot)
            put(j, slot).wait()
            g.wait()
            return 0

        lax.fori_loop(0, n_chunks - 1, body, 0, unroll=False)
        last = n_chunks - 1
        put(last, lax.rem(last, 2)).wait()

    return emb(idx, table)


_PB = 1024
_HP = 490 * _PB


def _pack_table(Wt):
    D, V = Wt.shape
    G = _HP // _PB
    last_b = (V - 1) // _PB

    def body(a_ref, b_ref, out_ref):
        out_ref[:, 0:D] = jnp.transpose(a_ref[...])
        out_ref[:, D:2 * D] = jnp.transpose(b_ref[...])

    return pl.pallas_call(
        body,
        grid=(G,),
        in_specs=[
            pl.BlockSpec((D, _PB), lambda g: (0, g)),
            pl.BlockSpec((D, _PB), lambda g: (0, jnp.minimum(g + G, last_b))),
        ],
        out_specs=pl.BlockSpec((_PB, 2 * D), lambda g: (g, 0)),
        out_shape=jax.ShapeDtypeStruct((_HP, 2 * D), jnp.float32),
    )(Wt, Wt)


def kernel(tokens, W_E):
    B, S = tokens.shape
    V, D = W_E.shape
    N = B * S
    chunk = 128
    n_chunks = N // (_NW * chunk)
    assert N == _NW * n_chunks * chunk
    t32 = tokens.astype(jnp.int32)
    idx = jnp.where(t32 < _HP, 2 * t32, 2 * (t32 - _HP) + 1)
    idx = idx.reshape(_NW, n_chunks, chunk)
    table_lin = _pack_table(W_E.T).reshape(2 * _HP, D)
    out = _emb_call(n_chunks, chunk, D, idx, table_lin)
    return out.reshape(B, S, D)


# speedup vs baseline: 1.4109x; 1.2617x over previous
import functools

import jax
import jax.numpy as jnp
from jax import lax
from jax.experimental import pallas as pl
from jax.experimental.pallas import tpu as pltpu
from jax.experimental.pallas import tpu_sc as plsc

_NC = 2
_NS = 16
_NW = _NC * _NS

_PB = 1024
_HP = 490 * _PB


def _pack_table(Wt):
    D, V = Wt.shape
    G = _HP // _PB
    last_b = (V - 1) // _PB

    def body(a_ref, b_ref, out_ref):
        eye = (lax.broadcasted_iota(jnp.int32, (D, D), 0)
               == lax.broadcasted_iota(jnp.int32, (D, D), 1)).astype(jnp.float32)
        dn = (((0,), (0,)), ((), ()))
        out_ref[:, 0:D] = lax.dot_general(
            a_ref[...], eye, dn, preferred_element_type=jnp.float32)
        out_ref[:, D:2 * D] = lax.dot_general(
            b_ref[...], eye, dn, preferred_element_type=jnp.float32)

    return pl.pallas_call(
        body,
        grid=(G,),
        in_specs=[
            pl.BlockSpec((D, _PB), lambda g: (0, g)),
            pl.BlockSpec((D, _PB), lambda g: (0, jnp.minimum(g + G, last_b))),
        ],
        out_specs=pl.BlockSpec((_PB, 2 * D), lambda g: (g, 0)),
        out_shape=jax.ShapeDtypeStruct((_HP, 2 * D), jnp.float32),
    )(Wt, Wt)


def _emb_call(n_chunks, chunk, D, B, idx, table):
    mesh = plsc.VectorSubcoreMesh(core_axis_name="c", subcore_axis_name="s")
    N = _NW * n_chunks * chunk
    HB = B // 2

    @functools.partial(
        pl.kernel,
        mesh=mesh,
        compiler_params=pltpu.CompilerParams(use_tc_tiling_on_sc=False),
        out_type=jax.ShapeDtypeStruct((N // 2, 2, D), jnp.float32),
        scratch_types=[
            pltpu.VMEM((n_chunks, chunk), jnp.int32),
            pltpu.VMEM((2, chunk, D), jnp.float32),
            pltpu.SemaphoreType.DMA,
            pltpu.SemaphoreType.DMA,
            pltpu.SemaphoreType.DMA,
        ],
    )
    def emb(idx_hbm, table_hbm, out_hbm, idx_v, rows_v, isem, gsem, osem):
        wid = lax.axis_index("s") * _NC + lax.axis_index("c")
        base = wid * (n_chunks * chunk)
        pltpu.async_copy(idx_hbm.at[wid], idx_v, isem).wait()

        def gather(j, slot):
            return pltpu.async_copy(
                table_hbm.at[idx_v.at[j]], rows_v.at[slot], gsem)

        def put(j, slot):
            n0 = base + j * chunk
            s = n0 // B
            rem = n0 - s * B
            h = rem // HB
            q0 = s * HB + rem - h * HB
            return pltpu.async_copy(
                rows_v.at[slot], out_hbm.at[pl.ds(q0, chunk), h], osem)

        gather(0, 0).wait()

        def body(j, _):
            slot = lax.rem(j, 2)
            g = gather(j + 1, 1 - slot)
            put(j, slot).wait()
            g.wait()
            return 0

        lax.fori_loop(0, n_chunks - 1, body, 0, unroll=False)
        last = n_chunks - 1
        put(last, lax.rem(last, 2)).wait()

    return emb(idx, table)


def _retile(x, S, B, D):
    HB = B // 2

    def body(in_ref, out_ref):
        xx = in_ref[0]
        eye = (lax.broadcasted_iota(jnp.int32, (D, D), 0)
               == lax.broadcasted_iota(jnp.int32, (D, D), 1)).astype(jnp.float32)
        dn = (((1,), (1,)), ((), ()))
        out_ref[0, :, 0:HB] = lax.dot_general(
            eye, xx[:, 0:D], dn, preferred_element_type=jnp.float32)
        out_ref[0, :, HB:B] = lax.dot_general(
            eye, xx[:, D:2 * D], dn, preferred_element_type=jnp.float32)

    return pl.pallas_call(
        body,
        grid=(S,),
        in_specs=[pl.BlockSpec((1, HB, 2 * D), lambda s: (s, 0, 0))],
        out_specs=pl.BlockSpec((1, D, B), lambda s: (s, 0, 0)),
        out_shape=jax.ShapeDtypeStruct((S, D, B), jnp.float32),
    )(x)


def kernel(tokens, W_E):
    B, S = tokens.shape
    V, D = W_E.shape
    N = B * S
    chunk = 128
    n_chunks = N // (_NW * chunk)
    assert N == _NW * n_chunks * chunk

    t32 = tokens.T.astype(jnp.int32)
    idx = jnp.where(t32 < _HP, 2 * t32, 2 * (t32 - _HP) + 1)
    idx = idx.reshape(_NW, n_chunks, chunk)

    table_lin = _pack_table(W_E.T).reshape(2 * _HP, D)
    out = _emb_call(n_chunks, chunk, D, B, idx, table_lin)
    out_t = _retile(out.reshape(S, B // 2, 2 * D), S, B, D)
    return jnp.transpose(out_t, (2, 0, 1))
